# baseline (speedup 1.0000x reference)
import functools

import jax
import jax.numpy as jnp
from jax import lax
from jax.experimental import pallas as pl
from jax.experimental.pallas import tpu as pltpu
from jax.experimental.pallas import tpu_sc as plsc

N = 10000
E = 160000
D = 364
DP = 384
B = 4096

NC = 2
NS = 16
CW = 128
KCH = DP // CW
CH = 128
EPW = 10240
NCH = EPW // CH
G = 20
NG = NCH // G
NSLAB = KCH + 1
NACC = 10112
STRIPE = NACC // NS


@functools.cache
def _sc_kernels():
    mesh = plsc.VectorSubcoreMesh(core_axis_name="c", subcore_axis_name="s")

    @functools.partial(
        pl.kernel,
        mesh=mesh,
        out_type=jax.ShapeDtypeStruct((NSLAB * NACC, CW), jnp.float32),
        scratch_types=[
            pltpu.VMEM_SHARED((NACC, CW), jnp.float32),
            pltpu.VMEM((G, CH), jnp.int32),
            pltpu.VMEM((G, CH), jnp.int32),
            pltpu.VMEM((2, CH, CW), jnp.float32),
            pltpu.SemaphoreType.DMA,
            pltpu.SemaphoreType.DMA,
            pltpu.SemaphoreType.DMA,
        ],
    )
    def _sc_edge_agg(tables, srcs, dsts, zeros, out, acc, src_v, dst_v,
                     rows_v, sem_i, sem_g0, sem_g1):
        c = lax.axis_index("c")
        s = lax.axis_index("s")

        def run_pass(k, slab, g0, ng):
            table = tables.at[k]

            def gather(j, slot, sem):
                return pltpu.make_async_copy(
                    table.at[src_v.at[j]], rows_v.at[slot], sem)

            def scatter_add(j, slot):
                pltpu.sync_copy(rows_v.at[slot], acc.at[dst_v.at[j]],
                                add=True)

            pltpu.sync_copy(zeros, acc.at[pl.ds(s * STRIPE, STRIPE)])
            plsc.subcore_barrier()
            for gg in range(ng):
                g = g0 + gg
                hs = pltpu.make_async_copy(srcs.at[s * NG + g], src_v,
                                           sem_i)
                hd = pltpu.make_async_copy(dsts.at[s * NG + g], dst_v,
                                           sem_i)
                hs.start()
                hd.start()
                hs.wait()
                hd.wait()
                gather(0, 0, sem_g0).start()
                gather(1, 1, sem_g1).start()

                def body(t, carry):
                    j0 = 2 * t
                    gather(j0, 0, sem_g0).wait()
                    scatter_add(j0, 0)
                    gather(j0 + 2, 0, sem_g0).start()
                    gather(j0 + 1, 1, sem_g1).wait()
                    scatter_add(j0 + 1, 1)
                    gather(j0 + 3, 1, sem_g1).start()
                    return carry

                lax.fori_loop(0, G // 2 - 1, body, 0)
                gather(G - 2, 0, sem_g0).wait()
                scatter_add(G - 2, 0)
                gather(G - 1, 1, sem_g1).wait()
                scatter_add(G - 1, 1)
            plsc.subcore_barrier()
            pltpu.sync_copy(acc.at[pl.ds(s * STRIPE, STRIPE)],
                            out.at[pl.ds(slab * NACC + s * STRIPE, STRIPE)])

        @pl.when(c == 0)
        def _sc0():
            run_pass(0, 0, 0, NG)
            run_pass(2, 2, 0, 1)

        @pl.when(c == 1)
        def _sc1():
            run_pass(1, 1, 0, NG)
            run_pass(2, 3, 1, NG - 1)

    @functools.partial(
        pl.kernel,
        mesh=mesh,
        out_type=jax.ShapeDtypeStruct((2 * B, DP), jnp.float32),
        scratch_types=[
            pltpu.VMEM((2, CH), jnp.int32),
            pltpu.VMEM((CH, DP), jnp.float32),
            pltpu.SemaphoreType.DMA,
        ],
    )
    def _sc_row_gather(table, idx, out, idx_v, rows_v, sem):
        c = lax.axis_index("c")
        s = lax.axis_index("s")
        w = c * NS + s
        pltpu.sync_copy(idx.at[w], idx_v)
        for j in range(2):
            pltpu.async_copy(table.at[idx_v.at[j]], rows_v, sem).wait()
            pltpu.sync_copy(rows_v, out.at[pl.ds(w * 2 * CH + j * CH, CH)])

    return _sc_edge_agg, _sc_row_gather


_R1 = 2000


def _chunkify_body(f, out):
    out[0] = f[:, :CW]
    out[1] = f[:, CW:2 * CW]
    out[2] = jnp.concatenate(
        [f[:, 2 * CW:], jnp.zeros((f.shape[0], DP - D), jnp.float32)],
        axis=1)


def _chunkify(feat):
    return pl.pallas_call(
        _chunkify_body, grid=(N // _R1,),
        in_specs=[pl.BlockSpec((_R1, D), lambda i: (i, 0))],
        out_specs=pl.BlockSpec((KCH, _R1, CW), lambda i: (0, i, 0)),
        out_shape=jax.ShapeDtypeStruct((KCH, N, CW), jnp.float32),
    )(feat)


def _gcn1_body(f, a0, a1, a2, a3, wf, wb0, wb1, wb2, b, out):
    y = (jnp.dot(f[...], wf[...], preferred_element_type=jnp.float32)
         + jnp.dot(a0[0], wb0[...], preferred_element_type=jnp.float32)
         + jnp.dot(a1[0], wb1[...], preferred_element_type=jnp.float32)
         + jnp.dot(a2[0] + a3[0], wb2[...],
                   preferred_element_type=jnp.float32))
    y = jnp.maximum(y + b[...], 0.0)
    out[0] = y[:, :CW]
    out[1] = y[:, CW:2 * CW]
    out[2] = y[:, 2 * CW:]


def _gcn2_body(x0, x1, x2, a0, a1, a2, a3, wf0, wf1, wf2, wb0, wb1, wb2, b,
               out):
    y = (jnp.dot(x0[0], wf0[...], preferred_element_type=jnp.float32)
         + jnp.dot(x1[0], wf1[...], preferred_element_type=jnp.float32)
         + jnp.dot(x2[0], wf2[...], preferred_element_type=jnp.float32)
         + jnp.dot(a0[0], wb0[...], preferred_element_type=jnp.float32)
         + jnp.dot(a1[0], wb1[...], preferred_element_type=jnp.float32)
         + jnp.dot(a2[0] + a3[0], wb2[...],
                   preferred_element_type=jnp.float32))
    out[...] = jnp.maximum(y + b[...], 0.0)


def _full(arr_shape):
    nd = len(arr_shape)
    return pl.BlockSpec(arr_shape, lambda i: (0,) * nd)


def _chunk_specs(n):
    return [pl.BlockSpec((1, _R1, CW), lambda i, k=k: (k, i, 0))
            for k in range(n)]


def _gcn_layer1(feat, agg4, wf, wbs, b):
    in_specs = ([pl.BlockSpec((_R1, D), lambda i: (i, 0))]
                + _chunk_specs(NSLAB)
                + [_full(wf.shape)] + [_full(w.shape) for w in wbs]
                + [_full(b.shape)])
    return pl.pallas_call(
        _gcn1_body, grid=(N // _R1,), in_specs=in_specs,
        out_specs=pl.BlockSpec((KCH, _R1, CW), lambda i: (0, i, 0)),
        out_shape=jax.ShapeDtypeStruct((KCH, N, CW), jnp.float32),
    )(feat, agg4, agg4, agg4, agg4, wf, *wbs, b)


def _gcn_layer2(x3, agg4, wfs, wbs, b):
    in_specs = (_chunk_specs(KCH) + _chunk_specs(NSLAB)
                + [_full(w.shape) for w in wfs]
                + [_full(w.shape) for w in wbs] + [_full(b.shape)])
    return pl.pallas_call(
        _gcn2_body, grid=(N // _R1,), in_specs=in_specs,
        out_specs=pl.BlockSpec((_R1, DP), lambda i: (i, 0)),
        out_shape=jax.ShapeDtypeStruct((N, DP), jnp.float32),
    )(x3, x3, x3, agg4, agg4, agg4, agg4, *wfs, *wbs, b)


_R3 = 2048


def _head_body(g, w3, b3, w4, b4, w5, b5, out):
    z = jnp.maximum(
        jnp.dot(g[...], w3[...], preferred_element_type=jnp.float32)
        + b3[...], 0.0)
    z = jnp.maximum(
        jnp.dot(z, w4[...], preferred_element_type=jnp.float32)
        + b4[...], 0.0)
    z = jnp.maximum(
        jnp.dot(z, w5[...], preferred_element_type=jnp.float32)
        + b5[...], 0.0)
    nrm = jnp.sqrt(jnp.sum(z * z, axis=1, keepdims=True))
    out[...] = z / jnp.maximum(nrm, 1e-12)


def _head(g, w3, b3, w4, b4, w5, b5):
    in_specs = [
        pl.BlockSpec((_R3, DP), lambda i: (i, 0)),
        _full(w3.shape), _full(b3.shape), _full(w4.shape), _full(b4.shape),
        _full(w5.shape), _full(b5.shape),
    ]
    return pl.pallas_call(
        _head_body, grid=(2 * B // _R3,), in_specs=in_specs,
        out_specs=pl.BlockSpec((_R3, 100), lambda i: (i, 0)),
        out_shape=jax.ShapeDtypeStruct((2 * B, 100), jnp.float32),
    )(g, w3, b3, w4, b4, w5, b5)


def kernel(features, edge_index, v1, v2, W1, b1, W2, b2, W3, b3, W4, b4,
           W5, b5):
    src = edge_index[0].reshape(NS, E // NS)
    dst = edge_index[1].reshape(NS, E // NS)
    src = jnp.pad(src, ((0, 0), (0, EPW - E // NS)))
    dst = jnp.pad(dst, ((0, 0), (0, EPW - E // NS)), constant_values=N)
    srcs = src.reshape(NS * NG, G, CH)
    dsts = dst.reshape(NS * NG, G, CH)
    zeros = jnp.zeros((STRIPE, CW), jnp.float32)

    w1f = jnp.pad(W1[:D], ((0, 0), (0, DP - D)))
    w1b = jnp.pad(W1[D:], ((0, DP - D), (0, DP - D)))
    w2f = jnp.pad(W2[:D], ((0, DP - D), (0, DP - D)))
    w2b = jnp.pad(W2[D:], ((0, DP - D), (0, DP - D)))
    w1bs = [w1b[k * CW:(k + 1) * CW] for k in range(KCH)]
    w2fs = [w2f[k * CW:(k + 1) * CW] for k in range(KCH)]
    w2bs = [w2b[k * CW:(k + 1) * CW] for k in range(KCH)]
    b1p = jnp.pad(b1, (0, DP - D)).reshape(1, DP)
    b2p = jnp.pad(b2, (0, DP - D)).reshape(1, DP)
    w3p = jnp.pad(W3, ((0, DP - D), (0, 0)))
    b3p = b3.reshape(1, D)
    b4p = b4.reshape(1, 200)
    b5p = b5.reshape(1, 100)

    sc_edge_agg, sc_row_gather = _sc_kernels()

    tables1 = _chunkify(features)
    agg1 = sc_edge_agg(tables1, srcs, dsts, zeros)
    x3 = _gcn_layer1(features, agg1.reshape(NSLAB, NACC, CW), w1f, w1bs,
                     b1p)
    agg2 = sc_edge_agg(x3, srcs, dsts, zeros)
    gcn = _gcn_layer2(x3, agg2.reshape(NSLAB, NACC, CW), w2fs, w2bs, b2p)

    v = jnp.concatenate([v1, v2]).reshape(NC * NS, 2, CH)
    g = sc_row_gather(gcn, v)
    z = _head(g, w3p, b3p, W4, b4p, W5, b5p)
    return z[:B], z[B:]


# speedup vs baseline: 1.0522x; 1.0522x over previous
import functools

import jax
import jax.numpy as jnp
from jax import lax
from jax.experimental import pallas as pl
from jax.experimental.pallas import tpu as pltpu
from jax.experimental.pallas import tpu_sc as plsc

N = 10000
E = 160000
D = 364
DP = 384
B = 4096

NC = 2
NS = 16
CW = 128
KCH = DP // CW
CH = 128
EPW = 10240
NCH = EPW // CH
G = 20
NG = NCH // G
NSLAB = KCH + 1
NACC = 10112
STRIPE = NACC // NS


@functools.cache
def _sc_kernels():
    mesh = plsc.VectorSubcoreMesh(core_axis_name="c", subcore_axis_name="s")

    @functools.partial(
        pl.kernel,
        mesh=mesh,
        out_type=jax.ShapeDtypeStruct((NSLAB * NACC, CW), jnp.float32),
        scratch_types=[
            pltpu.VMEM_SHARED((NACC, CW), jnp.float32),
            pltpu.VMEM((G, CH), jnp.int32),
            pltpu.VMEM((G, CH), jnp.int32),
            pltpu.VMEM((2, CH, CW), jnp.float32),
            pltpu.SemaphoreType.DMA,
            pltpu.SemaphoreType.DMA,
            pltpu.SemaphoreType.DMA,
        ],
    )
    def _sc_edge_agg(tables, srcs, dsts, zeros, out, acc, src_v, dst_v,
                     rows_v, sem_i, sem_g0, sem_g1):
        c = lax.axis_index("c")
        s = lax.axis_index("s")

        def run_pass(k, slab, g0, ng):
            table = tables.at[k]

            def gather(j, slot, sem):
                return pltpu.make_async_copy(
                    table.at[src_v.at[j]], rows_v.at[slot], sem)

            def scatter_add(j, slot):
                pltpu.sync_copy(rows_v.at[slot], acc.at[dst_v.at[j]],
                                add=True)

            pltpu.sync_copy(zeros, acc.at[pl.ds(s * STRIPE, STRIPE)])
            plsc.subcore_barrier()
            for gg in range(ng):
                g = g0 + gg
                hs = pltpu.make_async_copy(srcs.at[s * NG + g], src_v,
                                           sem_i)
                hd = pltpu.make_async_copy(dsts.at[s * NG + g], dst_v,
                                           sem_i)
                hs.start()
                hd.start()
                hs.wait()
                hd.wait()
                gather(0, 0, sem_g0).start()
                gather(1, 1, sem_g1).start()

                def body(t, carry):
                    j0 = 2 * t
                    gather(j0, 0, sem_g0).wait()
                    scatter_add(j0, 0)
                    gather(j0 + 2, 0, sem_g0).start()
                    gather(j0 + 1, 1, sem_g1).wait()
                    scatter_add(j0 + 1, 1)
                    gather(j0 + 3, 1, sem_g1).start()
                    return carry

                lax.fori_loop(0, G // 2 - 1, body, 0)
                gather(G - 2, 0, sem_g0).wait()
                scatter_add(G - 2, 0)
                gather(G - 1, 1, sem_g1).wait()
                scatter_add(G - 1, 1)
            plsc.subcore_barrier()
            pltpu.sync_copy(acc.at[pl.ds(s * STRIPE, STRIPE)],
                            out.at[pl.ds(slab * NACC + s * STRIPE, STRIPE)])

        @pl.when(c == 0)
        def _sc0():
            run_pass(0, 0, 0, NG)
            run_pass(2, 2, 0, NG - 1)

        @pl.when(c == 1)
        def _sc1():
            run_pass(1, 1, 0, NG)
            run_pass(2, 3, NG - 1, 1)

    @functools.partial(
        pl.kernel,
        mesh=mesh,
        out_type=jax.ShapeDtypeStruct((2 * B, DP), jnp.float32),
        scratch_types=[
            pltpu.VMEM((2, CH), jnp.int32),
            pltpu.VMEM((CH, DP), jnp.float32),
            pltpu.SemaphoreType.DMA,
        ],
    )
    def _sc_row_gather(table, idx, out, idx_v, rows_v, sem):
        c = lax.axis_index("c")
        s = lax.axis_index("s")
        w = c * NS + s
        pltpu.sync_copy(idx.at[w], idx_v)
        for j in range(2):
            pltpu.async_copy(table.at[idx_v.at[j]], rows_v, sem).wait()
            pltpu.sync_copy(rows_v, out.at[pl.ds(w * 2 * CH + j * CH, CH)])

    return _sc_edge_agg, _sc_row_gather


_R1 = 2000


def _chunkify_body(f, out):
    out[0] = f[:, :CW]
    out[1] = f[:, CW:2 * CW]
    out[2] = jnp.concatenate(
        [f[:, 2 * CW:], jnp.zeros((f.shape[0], DP - D), jnp.float32)],
        axis=1)


def _chunkify(feat):
    return pl.pallas_call(
        _chunkify_body, grid=(N // _R1,),
        in_specs=[pl.BlockSpec((_R1, D), lambda i: (i, 0))],
        out_specs=pl.BlockSpec((KCH, _R1, CW), lambda i: (0, i, 0)),
        out_shape=jax.ShapeDtypeStruct((KCH, N, CW), jnp.float32),
    )(feat)


def _gcn1_body(f, a0, a1, a2, a3, wf, wb0, wb1, wb2, b, out):
    y = (jnp.dot(f[...], wf[...], preferred_element_type=jnp.float32)
         + jnp.dot(a0[0], wb0[...], preferred_element_type=jnp.float32)
         + jnp.dot(a1[0], wb1[...], preferred_element_type=jnp.float32)
         + jnp.dot(a2[0] + a3[0], wb2[...],
                   preferred_element_type=jnp.float32))
    y = jnp.maximum(y + b[...], 0.0)
    out[0] = y[:, :CW]
    out[1] = y[:, CW:2 * CW]
    out[2] = y[:, 2 * CW:]


def _gcn2_body(x0, x1, x2, a0, a1, a2, a3, wf0, wf1, wf2, wb0, wb1, wb2, b,
               out):
    y = (jnp.dot(x0[0], wf0[...], preferred_element_type=jnp.float32)
         + jnp.dot(x1[0], wf1[...], preferred_element_type=jnp.float32)
         + jnp.dot(x2[0], wf2[...], preferred_element_type=jnp.float32)
         + jnp.dot(a0[0], wb0[...], preferred_element_type=jnp.float32)
         + jnp.dot(a1[0], wb1[...], preferred_element_type=jnp.float32)
         + jnp.dot(a2[0] + a3[0], wb2[...],
                   preferred_element_type=jnp.float32))
    out[...] = jnp.maximum(y + b[...], 0.0)


def _full(arr_shape):
    nd = len(arr_shape)
    return pl.BlockSpec(arr_shape, lambda i: (0,) * nd)


def _chunk_specs(n):
    return [pl.BlockSpec((1, _R1, CW), lambda i, k=k: (k, i, 0))
            for k in range(n)]


def _gcn_layer1(feat, agg4, wf, wbs, b):
    in_specs = ([pl.BlockSpec((_R1, D), lambda i: (i, 0))]
                + _chunk_specs(NSLAB)
                + [_full(wf.shape)] + [_full(w.shape) for w in wbs]
                + [_full(b.shape)])
    return pl.pallas_call(
        _gcn1_body, grid=(N // _R1,), in_specs=in_specs,
        out_specs=pl.BlockSpec((KCH, _R1, CW), lambda i: (0, i, 0)),
        out_shape=jax.ShapeDtypeStruct((KCH, N, CW), jnp.float32),
    )(feat, agg4, agg4, agg4, agg4, wf, *wbs, b)


def _gcn_layer2(x3, agg4, wfs, wbs, b):
    in_specs = (_chunk_specs(KCH) + _chunk_specs(NSLAB)
                + [_full(w.shape) for w in wfs]
                + [_full(w.shape) for w in wbs] + [_full(b.shape)])
    return pl.pallas_call(
        _gcn2_body, grid=(N // _R1,), in_specs=in_specs,
        out_specs=pl.BlockSpec((_R1, DP), lambda i: (i, 0)),
        out_shape=jax.ShapeDtypeStruct((N, DP), jnp.float32),
    )(x3, x3, x3, agg4, agg4, agg4, agg4, *wfs, *wbs, b)


_R3 = 2048


def _head_body(g, w3, b3, w4, b4, w5, b5, out):
    z = jnp.maximum(
        jnp.dot(g[...], w3[...], preferred_element_type=jnp.float32)
        + b3[...], 0.0)
    z = jnp.maximum(
        jnp.dot(z, w4[...], preferred_element_type=jnp.float32)
        + b4[...], 0.0)
    z = jnp.maximum(
        jnp.dot(z, w5[...], preferred_element_type=jnp.float32)
        + b5[...], 0.0)
    nrm = jnp.sqrt(jnp.sum(z * z, axis=1, keepdims=True))
    out[...] = z / jnp.maximum(nrm, 1e-12)


def _head(g, w3, b3, w4, b4, w5, b5):
    in_specs = [
        pl.BlockSpec((_R3, DP), lambda i: (i, 0)),
        _full(w3.shape), _full(b3.shape), _full(w4.shape), _full(b4.shape),
        _full(w5.shape), _full(b5.shape),
    ]
    return pl.pallas_call(
        _head_body, grid=(2 * B // _R3,), in_specs=in_specs,
        out_specs=pl.BlockSpec((_R3, 100), lambda i: (i, 0)),
        out_shape=jax.ShapeDtypeStruct((2 * B, 100), jnp.float32),
    )(g, w3, b3, w4, b4, w5, b5)


def kernel(features, edge_index, v1, v2, W1, b1, W2, b2, W3, b3, W4, b4,
           W5, b5):
    src = edge_index[0].reshape(NS, E // NS)
    dst = edge_index[1].reshape(NS, E // NS)
    src = jnp.pad(src, ((0, 0), (0, EPW - E // NS)))
    dst = jnp.pad(dst, ((0, 0), (0, EPW - E // NS)), constant_values=N)
    srcs = src.reshape(NS * NG, G, CH)
    dsts = dst.reshape(NS * NG, G, CH)
    zeros = jnp.zeros((STRIPE, CW), jnp.float32)

    w1f = jnp.pad(W1[:D], ((0, 0), (0, DP - D)))
    w1b = jnp.pad(W1[D:], ((0, DP - D), (0, DP - D)))
    w2f = jnp.pad(W2[:D], ((0, DP - D), (0, DP - D)))
    w2b = jnp.pad(W2[D:], ((0, DP - D), (0, DP - D)))
    w1bs = [w1b[k * CW:(k + 1) * CW] for k in range(KCH)]
    w2fs = [w2f[k * CW:(k + 1) * CW] for k in range(KCH)]
    w2bs = [w2b[k * CW:(k + 1) * CW] for k in range(KCH)]
    b1p = jnp.pad(b1, (0, DP - D)).reshape(1, DP)
    b2p = jnp.pad(b2, (0, DP - D)).reshape(1, DP)
    w3p = jnp.pad(W3, ((0, DP - D), (0, 0)))
    b3p = b3.reshape(1, D)
    b4p = b4.reshape(1, 200)
    b5p = b5.reshape(1, 100)

    sc_edge_agg, sc_row_gather = _sc_kernels()

    tables1 = _chunkify(features)
    agg1 = sc_edge_agg(tables1, srcs, dsts, zeros)
    x3 = _gcn_layer1(features, agg1.reshape(NSLAB, NACC, CW), w1f, w1bs,
                     b1p)
    agg2 = sc_edge_agg(x3, srcs, dsts, zeros)
    gcn = _gcn_layer2(x3, agg2.reshape(NSLAB, NACC, CW), w2fs, w2bs, b2p)

    v = jnp.concatenate([v1, v2]).reshape(NC * NS, 2, CH)
    g = sc_row_gather(gcn, v)
    z = _head(g, w3p, b3p, W4, b4p, W5, b5p)
    return z[:B], z[B:]


# speedup vs baseline: 1.0621x; 1.0094x over previous
import functools

import jax
import jax.numpy as jnp
from jax import lax
from jax.experimental import pallas as pl
from jax.experimental.pallas import tpu as pltpu
from jax.experimental.pallas import tpu_sc as plsc

N = 10000
E = 160000
D = 364
DP = 384
B = 4096

NC = 2
NS = 16
CW = 128
KCH = DP // CW
CH = 128
EPW = 10240
NCH = EPW // CH
G = 20
NG = NCH // G
NSLAB = KCH + 1
NACC = 10112
STRIPE = NACC // NS


@functools.cache
def _sc_kernels():
    mesh = plsc.VectorSubcoreMesh(core_axis_name="c", subcore_axis_name="s")

    @functools.partial(
        pl.kernel,
        mesh=mesh,
        out_type=jax.ShapeDtypeStruct((NSLAB * NACC, CW), jnp.float32),
        scratch_types=[
            pltpu.VMEM_SHARED((NACC, CW), jnp.float32),
            pltpu.VMEM((G, CH), jnp.int32),
            pltpu.VMEM((G, CH), jnp.int32),
            pltpu.VMEM((2, CH, CW), jnp.float32),
            pltpu.VMEM((80, CW), jnp.float32),
            pltpu.SemaphoreType.DMA,
            pltpu.SemaphoreType.DMA,
            pltpu.SemaphoreType.DMA,
        ],
    )
    def _sc_edge_agg(tables, srcs, dsts, zeros, out, acc, src_v, dst_v,
                     rows_v, zbuf, sem_i, sem_g0, sem_g1):
        c = lax.axis_index("c")
        s = lax.axis_index("s")
        pltpu.sync_copy(zeros, zbuf)

        def run_pass(k, slab, g0, ng):
            table = tables.at[k]

            def gather(j, slot, sem):
                return pltpu.make_async_copy(
                    table.at[src_v.at[j]], rows_v.at[slot], sem)

            def scatter_add(j, slot):
                pltpu.sync_copy(rows_v.at[slot], acc.at[dst_v.at[j]],
                                add=True)

            for z in range(7):
                pltpu.sync_copy(zbuf, acc.at[pl.ds(s * STRIPE + z * 80, 80)])
            pltpu.sync_copy(zbuf.at[pl.ds(0, 72)],
                            acc.at[pl.ds(s * STRIPE + 560, 72)])
            plsc.subcore_barrier()
            for gg in range(ng):
                g = g0 + gg
                hs = pltpu.make_async_copy(srcs.at[s * NG + g], src_v,
                                           sem_i)
                hd = pltpu.make_async_copy(dsts.at[s * NG + g], dst_v,
                                           sem_i)
                hs.start()
                hd.start()
                hs.wait()
                hd.wait()
                gather(0, 0, sem_g0).start()
                gather(1, 1, sem_g1).start()

                def body(t, carry):
                    j0 = 2 * t
                    gather(j0, 0, sem_g0).wait()
                    scatter_add(j0, 0)
                    gather(j0 + 2, 0, sem_g0).start()
                    gather(j0 + 1, 1, sem_g1).wait()
                    scatter_add(j0 + 1, 1)
                    gather(j0 + 3, 1, sem_g1).start()
                    return carry

                lax.fori_loop(0, G // 2 - 1, body, 0)
                gather(G - 2, 0, sem_g0).wait()
                scatter_add(G - 2, 0)
                gather(G - 1, 1, sem_g1).wait()
                scatter_add(G - 1, 1)
            plsc.subcore_barrier()
            pltpu.sync_copy(acc.at[pl.ds(s * STRIPE, STRIPE)],
                            out.at[pl.ds(slab * NACC + s * STRIPE, STRIPE)])

        @pl.when(c == 0)
        def _sc0():
            run_pass(0, 0, 0, NG)
            run_pass(2, 2, 0, NG - 1)

        @pl.when(c == 1)
        def _sc1():
            run_pass(1, 1, 0, NG)
            run_pass(2, 3, NG - 1, 1)

    @functools.partial(
        pl.kernel,
        mesh=mesh,
        out_type=jax.ShapeDtypeStruct((2 * B, DP), jnp.float32),
        scratch_types=[
            pltpu.VMEM((2, CH), jnp.int32),
            pltpu.VMEM((CH, DP), jnp.float32),
            pltpu.SemaphoreType.DMA,
        ],
    )
    def _sc_row_gather(table, idx, out, idx_v, rows_v, sem):
        c = lax.axis_index("c")
        s = lax.axis_index("s")
        w = c * NS + s
        pltpu.sync_copy(idx.at[w], idx_v)
        for j in range(2):
            pltpu.async_copy(table.at[idx_v.at[j]], rows_v, sem).wait()
            pltpu.sync_copy(rows_v, out.at[pl.ds(w * 2 * CH + j * CH, CH)])

    return _sc_edge_agg, _sc_row_gather


_R1 = 2000


def _chunkify_body(f, out):
    out[0] = f[:, :CW]
    out[1] = f[:, CW:2 * CW]
    out[2] = jnp.concatenate(
        [f[:, 2 * CW:], jnp.zeros((f.shape[0], DP - D), jnp.float32)],
        axis=1)


def _chunkify(feat):
    return pl.pallas_call(
        _chunkify_body, grid=(N // _R1,),
        in_specs=[pl.BlockSpec((_R1, D), lambda i: (i, 0))],
        out_specs=pl.BlockSpec((KCH, _R1, CW), lambda i: (0, i, 0)),
        out_shape=jax.ShapeDtypeStruct((KCH, N, CW), jnp.float32),
    )(feat)


def _gcn1_body(f, a0, a1, a2, a3, wf, wb0, wb1, wb2, b, out):
    y = (jnp.dot(f[...], wf[...], preferred_element_type=jnp.float32)
         + jnp.dot(a0[0], wb0[...], preferred_element_type=jnp.float32)
         + jnp.dot(a1[0], wb1[...], preferred_element_type=jnp.float32)
         + jnp.dot(a2[0] + a3[0], wb2[...],
                   preferred_element_type=jnp.float32))
    y = jnp.maximum(y + b[...], 0.0)
    out[0] = y[:, :CW]
    out[1] = y[:, CW:2 * CW]
    out[2] = y[:, 2 * CW:]


def _gcn2_body(x0, x1, x2, a0, a1, a2, a3, wf0, wf1, wf2, wb0, wb1, wb2, b,
               out):
    y = (jnp.dot(x0[0], wf0[...], preferred_element_type=jnp.float32)
         + jnp.dot(x1[0], wf1[...], preferred_element_type=jnp.float32)
         + jnp.dot(x2[0], wf2[...], preferred_element_type=jnp.float32)
         + jnp.dot(a0[0], wb0[...], preferred_element_type=jnp.float32)
         + jnp.dot(a1[0], wb1[...], preferred_element_type=jnp.float32)
         + jnp.dot(a2[0] + a3[0], wb2[...],
                   preferred_element_type=jnp.float32))
    out[...] = jnp.maximum(y + b[...], 0.0)


def _full(arr_shape):
    nd = len(arr_shape)
    return pl.BlockSpec(arr_shape, lambda i: (0,) * nd)


def _chunk_specs(n):
    return [pl.BlockSpec((1, _R1, CW), lambda i, k=k: (k, i, 0))
            for k in range(n)]


def _gcn_layer1(feat, agg4, wf, wbs, b):
    in_specs = ([pl.BlockSpec((_R1, D), lambda i: (i, 0))]
                + _chunk_specs(NSLAB)
                + [_full(wf.shape)] + [_full(w.shape) for w in wbs]
                + [_full(b.shape)])
    return pl.pallas_call(
        _gcn1_body, grid=(N // _R1,), in_specs=in_specs,
        out_specs=pl.BlockSpec((KCH, _R1, CW), lambda i: (0, i, 0)),
        out_shape=jax.ShapeDtypeStruct((KCH, N, CW), jnp.float32),
    )(feat, agg4, agg4, agg4, agg4, wf, *wbs, b)


def _gcn_layer2(x3, agg4, wfs, wbs, b):
    in_specs = (_chunk_specs(KCH) + _chunk_specs(NSLAB)
                + [_full(w.shape) for w in wfs]
                + [_full(w.shape) for w in wbs] + [_full(b.shape)])
    return pl.pallas_call(
        _gcn2_body, grid=(N // _R1,), in_specs=in_specs,
        out_specs=pl.BlockSpec((_R1, DP), lambda i: (i, 0)),
        out_shape=jax.ShapeDtypeStruct((N, DP), jnp.float32),
    )(x3, x3, x3, agg4, agg4, agg4, agg4, *wfs, *wbs, b)


_R3 = 2048


def _head_body(g, w3, b3, w4, b4, w5, b5, out):
    z = jnp.maximum(
        jnp.dot(g[...], w3[...], preferred_element_type=jnp.float32)
        + b3[...], 0.0)
    z = jnp.maximum(
        jnp.dot(z, w4[...], preferred_element_type=jnp.float32)
        + b4[...], 0.0)
    z = jnp.maximum(
        jnp.dot(z, w5[...], preferred_element_type=jnp.float32)
        + b5[...], 0.0)
    nrm = jnp.sqrt(jnp.sum(z * z, axis=1, keepdims=True))
    out[...] = z / jnp.maximum(nrm, 1e-12)


def _head(g, w3, b3, w4, b4, w5, b5):
    in_specs = [
        pl.BlockSpec((_R3, DP), lambda i: (i, 0)),
        _full(w3.shape), _full(b3.shape), _full(w4.shape), _full(b4.shape),
        _full(w5.shape), _full(b5.shape),
    ]
    return pl.pallas_call(
        _head_body, grid=(2 * B // _R3,), in_specs=in_specs,
        out_specs=pl.BlockSpec((_R3, 100), lambda i: (i, 0)),
        out_shape=jax.ShapeDtypeStruct((2 * B, 100), jnp.float32),
    )(g, w3, b3, w4, b4, w5, b5)


def kernel(features, edge_index, v1, v2, W1, b1, W2, b2, W3, b3, W4, b4,
           W5, b5):
    src = edge_index[0].reshape(NS, E // NS)
    dst = edge_index[1].reshape(NS, E // NS)
    src = jnp.pad(src, ((0, 0), (0, EPW - E // NS)))
    dst = jnp.pad(dst, ((0, 0), (0, EPW - E // NS)), constant_values=N)
    srcs = src.reshape(NS * NG, G, CH)
    dsts = dst.reshape(NS * NG, G, CH)
    zeros = jnp.zeros((80, CW), jnp.float32)

    w1f = jnp.pad(W1[:D], ((0, 0), (0, DP - D)))
    w1b = jnp.pad(W1[D:], ((0, DP - D), (0, DP - D)))
    w2f = jnp.pad(W2[:D], ((0, DP - D), (0, DP - D)))
    w2b = jnp.pad(W2[D:], ((0, DP - D), (0, DP - D)))
    w1bs = [w1b[k * CW:(k + 1) * CW] for k in range(KCH)]
    w2fs = [w2f[k * CW:(k + 1) * CW] for k in range(KCH)]
    w2bs = [w2b[k * CW:(k + 1) * CW] for k in range(KCH)]
    b1p = jnp.pad(b1, (0, DP - D)).reshape(1, DP)
    b2p = jnp.pad(b2, (0, DP - D)).reshape(1, DP)
    w3p = jnp.pad(W3, ((0, DP - D), (0, 0)))
    b3p = b3.reshape(1, D)
    b4p = b4.reshape(1, 200)
    b5p = b5.reshape(1, 100)

    sc_edge_agg, sc_row_gather = _sc_kernels()

    tables1 = _chunkify(features)
    agg1 = sc_edge_agg(tables1, srcs, dsts, zeros)
    x3 = _gcn_layer1(features, agg1.reshape(NSLAB, NACC, CW), w1f, w1bs,
                     b1p)
    agg2 = sc_edge_agg(x3, srcs, dsts, zeros)
    gcn = _gcn_layer2(x3, agg2.reshape(NSLAB, NACC, CW), w2fs, w2bs, b2p)

    v = jnp.concatenate([v1, v2]).reshape(NC * NS, 2, CH)
    g = sc_row_gather(gcn, v)
    z = _head(g, w3p, b3p, W4, b4p, W5, b5p)
    return z[:B], z[B:]


# speedup vs baseline: 1.0630x; 1.0009x over previous
import functools

import jax
import jax.numpy as jnp
from jax import lax
from jax.experimental import pallas as pl
from jax.experimental.pallas import tpu as pltpu
from jax.experimental.pallas import tpu_sc as plsc

N = 10000
E = 160000
D = 364
DP = 384
B = 4096

NC = 2
NS = 16
CW = 128
KCH = DP // CW
CH = 128
EPW = 10240
NCH = EPW // CH
G = 20
NG = NCH // G
NSLAB = KCH + 1
NACC = 10112
STRIPE = NACC // NS


@functools.cache
def _sc_kernels():
    mesh = plsc.VectorSubcoreMesh(core_axis_name="c", subcore_axis_name="s")

    @functools.partial(
        pl.kernel,
        mesh=mesh,
        out_type=jax.ShapeDtypeStruct((NSLAB * NACC, CW), jnp.float32),
        scratch_types=[
            pltpu.VMEM_SHARED((NACC, CW), jnp.float32),
            pltpu.VMEM((G, CH), jnp.int32),
            pltpu.VMEM((G, CH), jnp.int32),
            pltpu.VMEM((2, CH, CW), jnp.float32),
            pltpu.VMEM((80, CW), jnp.float32),
            pltpu.SemaphoreType.DMA,
            pltpu.SemaphoreType.DMA,
            pltpu.SemaphoreType.DMA,
        ],
    )
    def _sc_edge_agg(tables, srcs, dsts, zeros, out, acc, src_v, dst_v,
                     rows_v, zbuf, sem_i, sem_g0, sem_g1):
        c = lax.axis_index("c")
        s = lax.axis_index("s")
        pltpu.sync_copy(zeros, zbuf)

        def run_pass(k, slab, g0, ng):
            table = tables.at[k]

            def gather(j, slot, sem):
                return pltpu.make_async_copy(
                    table.at[src_v.at[j]], rows_v.at[slot], sem)

            def scatter_add(j, slot):
                pltpu.sync_copy(rows_v.at[slot], acc.at[dst_v.at[j]],
                                add=True)

            for z in range(7):
                pltpu.sync_copy(zbuf, acc.at[pl.ds(s * STRIPE + z * 80, 80)])
            pltpu.sync_copy(zbuf.at[pl.ds(0, 72)],
                            acc.at[pl.ds(s * STRIPE + 560, 72)])
            plsc.subcore_barrier()

            def group_body(g, carry):
                hs = pltpu.make_async_copy(srcs.at[s * NG + g], src_v,
                                           sem_i)
                hd = pltpu.make_async_copy(dsts.at[s * NG + g], dst_v,
                                           sem_i)
                hs.start()
                hd.start()
                hs.wait()
                hd.wait()
                gather(0, 0, sem_g0).start()
                gather(1, 1, sem_g1).start()

                def body(t, carry2):
                    j0 = 2 * t
                    gather(j0, 0, sem_g0).wait()
                    scatter_add(j0, 0)
                    gather(j0 + 2, 0, sem_g0).start()
                    gather(j0 + 1, 1, sem_g1).wait()
                    scatter_add(j0 + 1, 1)
                    gather(j0 + 3, 1, sem_g1).start()
                    return carry2

                lax.fori_loop(0, G // 2 - 1, body, 0)
                gather(G - 2, 0, sem_g0).wait()
                scatter_add(G - 2, 0)
                gather(G - 1, 1, sem_g1).wait()
                scatter_add(G - 1, 1)
                return carry

            lax.fori_loop(g0, g0 + ng, group_body, 0)
            plsc.subcore_barrier()
            pltpu.sync_copy(acc.at[pl.ds(s * STRIPE, STRIPE)],
                            out.at[pl.ds(slab * NACC + s * STRIPE, STRIPE)])

        @pl.when(c == 0)
        def _sc0():
            run_pass(0, 0, 0, NG)
            run_pass(2, 2, 0, NG - 1)

        @pl.when(c == 1)
        def _sc1():
            run_pass(1, 1, 0, NG)
            run_pass(2, 3, NG - 1, 1)

    @functools.partial(
        pl.kernel,
        mesh=mesh,
        out_type=jax.ShapeDtypeStruct((2 * B, DP), jnp.float32),
        scratch_types=[
            pltpu.VMEM((2, CH), jnp.int32),
            pltpu.VMEM((CH, DP), jnp.float32),
            pltpu.SemaphoreType.DMA,
        ],
    )
    def _sc_row_gather(table, idx, out, idx_v, rows_v, sem):
        c = lax.axis_index("c")
        s = lax.axis_index("s")
        w = c * NS + s
        pltpu.sync_copy(idx.at[w], idx_v)
        for j in range(2):
            pltpu.async_copy(table.at[idx_v.at[j]], rows_v, sem).wait()
            pltpu.sync_copy(rows_v, out.at[pl.ds(w * 2 * CH + j * CH, CH)])

    return _sc_edge_agg, _sc_row_gather


_R1 = 2000


def _chunkify_body(f, out):
    out[0] = f[:, :CW]
    out[1] = f[:, CW:2 * CW]
    out[2] = jnp.concatenate(
        [f[:, 2 * CW:], jnp.zeros((f.shape[0], DP - D), jnp.float32)],
        axis=1)


def _chunkify(feat):
    return pl.pallas_call(
        _chunkify_body, grid=(N // _R1,),
        in_specs=[pl.BlockSpec((_R1, D), lambda i: (i, 0))],
        out_specs=pl.BlockSpec((KCH, _R1, CW), lambda i: (0, i, 0)),
        out_shape=jax.ShapeDtypeStruct((KCH, N, CW), jnp.float32),
    )(feat)


def _gcn1_body(f, a0, a1, a2, a3, wf, wb0, wb1, wb2, b, out):
    y = (jnp.dot(f[...], wf[...], preferred_element_type=jnp.float32)
         + jnp.dot(a0[0], wb0[...], preferred_element_type=jnp.float32)
         + jnp.dot(a1[0], wb1[...], preferred_element_type=jnp.float32)
         + jnp.dot(a2[0] + a3[0], wb2[...],
                   preferred_element_type=jnp.float32))
    y = jnp.maximum(y + b[...], 0.0)
    out[0] = y[:, :CW]
    out[1] = y[:, CW:2 * CW]
    out[2] = y[:, 2 * CW:]


def _gcn2_body(x0, x1, x2, a0, a1, a2, a3, wf0, wf1, wf2, wb0, wb1, wb2, b,
               out):
    y = (jnp.dot(x0[0], wf0[...], preferred_element_type=jnp.float32)
         + jnp.dot(x1[0], wf1[...], preferred_element_type=jnp.float32)
         + jnp.dot(x2[0], wf2[...], preferred_element_type=jnp.float32)
         + jnp.dot(a0[0], wb0[...], preferred_element_type=jnp.float32)
         + jnp.dot(a1[0], wb1[...], preferred_element_type=jnp.float32)
         + jnp.dot(a2[0] + a3[0], wb2[...],
                   preferred_element_type=jnp.float32))
    out[...] = jnp.maximum(y + b[...], 0.0)


def _full(arr_shape):
    nd = len(arr_shape)
    return pl.BlockSpec(arr_shape, lambda i: (0,) * nd)


def _chunk_specs(n):
    return [pl.BlockSpec((1, _R1, CW), lambda i, k=k: (k, i, 0))
            for k in range(n)]


def _gcn_layer1(feat, agg4, wf, wbs, b):
    in_specs = ([pl.BlockSpec((_R1, D), lambda i: (i, 0))]
                + _chunk_specs(NSLAB)
                + [_full(wf.shape)] + [_full(w.shape) for w in wbs]
                + [_full(b.shape)])
    return pl.pallas_call(
        _gcn1_body, grid=(N // _R1,), in_specs=in_specs,
        out_specs=pl.BlockSpec((KCH, _R1, CW), lambda i: (0, i, 0)),
        out_shape=jax.ShapeDtypeStruct((KCH, N, CW), jnp.float32),
    )(feat, agg4, agg4, agg4, agg4, wf, *wbs, b)


def _gcn_layer2(x3, agg4, wfs, wbs, b):
    in_specs = (_chunk_specs(KCH) + _chunk_specs(NSLAB)
                + [_full(w.shape) for w in wfs]
                + [_full(w.shape) for w in wbs] + [_full(b.shape)])
    return pl.pallas_call(
        _gcn2_body, grid=(N // _R1,), in_specs=in_specs,
        out_specs=pl.BlockSpec((_R1, DP), lambda i: (i, 0)),
        out_shape=jax.ShapeDtypeStruct((N, DP), jnp.float32),
    )(x3, x3, x3, agg4, agg4, agg4, agg4, *wfs, *wbs, b)


_R3 = 2048


def _head_body(g, w3, b3, w4, b4, w5, b5, out):
    z = jnp.maximum(
        jnp.dot(g[...], w3[...], preferred_element_type=jnp.float32)
        + b3[...], 0.0)
    z = jnp.maximum(
        jnp.dot(z, w4[...], preferred_element_type=jnp.float32)
        + b4[...], 0.0)
    z = jnp.maximum(
        jnp.dot(z, w5[...], preferred_element_type=jnp.float32)
        + b5[...], 0.0)
    nrm = jnp.sqrt(jnp.sum(z * z, axis=1, keepdims=True))
    out[...] = z / jnp.maximum(nrm, 1e-12)


def _head(g, w3, b3, w4, b4, w5, b5):
    in_specs = [
        pl.BlockSpec((_R3, DP), lambda i: (i, 0)),
        _full(w3.shape), _full(b3.shape), _full(w4.shape), _full(b4.shape),
        _full(w5.shape), _full(b5.shape),
    ]
    return pl.pallas_call(
        _head_body, grid=(2 * B // _R3,), in_specs=in_specs,
        out_specs=pl.BlockSpec((_R3, 100), lambda i: (i, 0)),
        out_shape=jax.ShapeDtypeStruct((2 * B, 100), jnp.float32),
    )(g, w3, b3, w4, b4, w5, b5)


def kernel(features, edge_index, v1, v2, W1, b1, W2, b2, W3, b3, W4, b4,
           W5, b5):
    src = edge_index[0].reshape(NS, E // NS)
    dst = edge_index[1].reshape(NS, E // NS)
    src = jnp.pad(src, ((0, 0), (0, EPW - E // NS)))
    dst = jnp.pad(dst, ((0, 0), (0, EPW - E // NS)), constant_values=N)
    srcs = src.reshape(NS * NG, G, CH)
    dsts = dst.reshape(NS * NG, G, CH)
    zeros = jnp.zeros((80, CW), jnp.float32)

    w1f = jnp.pad(W1[:D], ((0, 0), (0, DP - D)))
    w1b = jnp.pad(W1[D:], ((0, DP - D), (0, DP - D)))
    w2f = jnp.pad(W2[:D], ((0, DP - D), (0, DP - D)))
    w2b = jnp.pad(W2[D:], ((0, DP - D), (0, DP - D)))
    w1bs = [w1b[k * CW:(k + 1) * CW] for k in range(KCH)]
    w2fs = [w2f[k * CW:(k + 1) * CW] for k in range(KCH)]
    w2bs = [w2b[k * CW:(k + 1) * CW] for k in range(KCH)]
    b1p = jnp.pad(b1, (0, DP - D)).reshape(1, DP)
    b2p = jnp.pad(b2, (0, DP - D)).reshape(1, DP)
    w3p = jnp.pad(W3, ((0, DP - D), (0, 0)))
    b3p = b3.reshape(1, D)
    b4p = b4.reshape(1, 200)
    b5p = b5.reshape(1, 100)

    sc_edge_agg, sc_row_gather = _sc_kernels()

    tables1 = _chunkify(features)
    agg1 = sc_edge_agg(tables1, srcs, dsts, zeros)
    x3 = _gcn_layer1(features, agg1.reshape(NSLAB, NACC, CW), w1f, w1bs,
                     b1p)
    agg2 = sc_edge_agg(x3, srcs, dsts, zeros)
    gcn = _gcn_layer2(x3, agg2.reshape(NSLAB, NACC, CW), w2fs, w2bs, b2p)

    v = jnp.concatenate([v1, v2]).reshape(NC * NS, 2, CH)
    g = sc_row_gather(gcn, v)
    z = _head(g, w3p, b3p, W4, b4p, W5, b5p)
    return z[:B], z[B:]
